# Initial kernel scaffold; baseline (speedup 1.0000x reference)
import functools

import jax
import jax.numpy as jnp
from jax import lax
from jax.experimental import pallas as pl
from jax.experimental.pallas import tpu as pltpu
from jax.experimental.pallas import tpu_sc as plsc

NC = 2
NS = 16
LANES = 16
CH = 80


@functools.lru_cache(maxsize=None)
def _sc_neighbor_sum(n_pad: int, d: int, e: int, compute_deg: bool):
    nw = NC * NS
    epw = e // nw
    assert epw * nw == e and epw % CH == 0
    nch = epw // CH
    rows_per_tile = n_pad // NS
    assert rows_per_tile * NS == n_pad and rows_per_tile % 2 == 0
    half = rows_per_tile // 2
    zrows = 64
    assert rows_per_tile % zrows == 0
    deg_rows = n_pad // 128
    assert deg_rows * 128 == n_pad and deg_rows % LANES == 0

    out_type = [jax.ShapeDtypeStruct((NC, n_pad, d), jnp.float32)]
    scratch = [
        pltpu.VMEM((CH,), jnp.int32),
        pltpu.VMEM((CH,), jnp.int32),
        pltpu.VMEM((CH, d), jnp.float32),
        pltpu.VMEM((zrows, d), jnp.float32),
        pltpu.VMEM((half, d), jnp.float32),
        pltpu.VMEM_SHARED((n_pad, d), jnp.float32),
        pltpu.SemaphoreType.DMA,
    ]
    if compute_deg:
        out_type.append(jax.ShapeDtypeStruct((NC, deg_rows, 128), jnp.float32))
        scratch += [
            pltpu.VMEM((deg_rows, 128), jnp.float32),
            pltpu.VMEM((deg_rows,), jnp.int32),
            pltpu.VMEM_SHARED((deg_rows, 128), jnp.float32),
        ]

    mesh = plsc.VectorSubcoreMesh(
        core_axis_name="c", subcore_axis_name="s",
        num_cores=NC, num_subcores=NS)

    @functools.partial(pl.kernel, out_type=tuple(out_type), mesh=mesh,
                       scratch_types=scratch)
    def sc_kernel(h_hbm, src_hbm, dst_hbm, sum_out, *rest):
        if compute_deg:
            (deg_out, src_v, dst_v, rows_v, zbuf, obuf, acc_sh, sem,
             hist_v, ident_v, deg_sh) = rest
        else:
            src_v, dst_v, rows_v, zbuf, obuf, acc_sh, sem = rest
        c = lax.axis_index("c")
        s = lax.axis_index("s")
        wid = c * NS + s
        zvec = jnp.zeros((LANES,), jnp.float32)

        def zb_body(i, carry):
            zbuf[i // 8, pl.ds((i % 8) * LANES, LANES)] = zvec
            return carry
        lax.fori_loop(0, zrows * (d // LANES), zb_body, None)

        def za_body(j, carry):
            pltpu.sync_copy(
                zbuf, acc_sh.at[pl.ds(s * rows_per_tile + j * zrows, zrows)])
            return carry
        lax.fori_loop(0, rows_per_tile // zrows, za_body, None)

        if compute_deg:
            def zh_body(i, carry):
                hist_v[i // 8, pl.ds((i % 8) * LANES, LANES)] = zvec
                return carry
            lax.fori_loop(0, deg_rows * 8, zh_body, None)
            iota = lax.broadcasted_iota(jnp.int32, (LANES,), 0)
            for g in range(deg_rows // LANES):
                ident_v[pl.ds(g * LANES, LANES)] = iota + g * LANES

            @pl.when(s == 0)
            def _zero_deg_sh():
                def zd_body(j, carry):
                    pltpu.sync_copy(zbuf.at[pl.ds(0, LANES)],
                                    deg_sh.at[pl.ds(j * LANES, LANES)])
                    return carry
                lax.fori_loop(0, deg_rows // LANES, zd_body, None)

        plsc.subcore_barrier()

        ones = jnp.full((LANES,), 1.0, jnp.float32)
        ebase = wid * epw

        def edge_body(i, carry):
            off = ebase + i * CH
            pltpu.sync_copy(src_hbm.at[pl.ds(off, CH)], src_v)
            pltpu.sync_copy(dst_hbm.at[pl.ds(off, CH)], dst_v)
            pltpu.async_copy(h_hbm.at[src_v], rows_v, sem).wait()
            if compute_deg:
                for g in range(CH // LANES):
                    idx = dst_v[pl.ds(g * LANES, LANES)]
                    plsc.addupdate_scatter(
                        hist_v,
                        [jnp.right_shift(idx, 7), jnp.bitwise_and(idx, 127)],
                        ones)
            pltpu.sync_copy(rows_v, acc_sh.at[dst_v], add=True)
            return carry
        lax.fori_loop(0, nch, edge_body, None)

        if compute_deg:
            pltpu.sync_copy(hist_v, deg_sh.at[ident_v], add=True)

        plsc.subcore_barrier()

        for j in range(2):
            r0 = s * rows_per_tile + j * half
            pltpu.sync_copy(acc_sh.at[pl.ds(r0, half)], obuf)
            pltpu.sync_copy(obuf, sum_out.at[c, pl.ds(r0, half)])

        if compute_deg:
            @pl.when(s == 0)
            def _deg_out():
                pltpu.sync_copy(deg_sh, hist_v)
                pltpu.sync_copy(hist_v, deg_out.at[c])

    return sc_kernel


def _tc_body(h_ref, p_ref, deg_ref, ws_ref, wn_ref, o_ref):
    deg = jnp.maximum(deg_ref[0] + deg_ref[1], 1.0)
    m = (p_ref[0] + p_ref[1]) / deg
    dn = (((1,), (1,)), ((), ()))
    acc = lax.dot_general(h_ref[...], ws_ref[...], dn,
                          preferred_element_type=jnp.float32,
                          precision=lax.Precision.HIGHEST)
    acc = acc + lax.dot_general(m, wn_ref[...], dn,
                                preferred_element_type=jnp.float32,
                                precision=lax.Precision.HIGHEST)
    o_ref[...] = jnp.maximum(acc, 0.0)


@functools.lru_cache(maxsize=None)
def _tc_layer(n_pad: int, d: int):
    bm = 1024
    assert n_pad % bm == 0
    return pl.pallas_call(
        _tc_body,
        grid=(n_pad // bm,),
        in_specs=[
            pl.BlockSpec((bm, d), lambda i: (i, 0)),
            pl.BlockSpec((NC, bm, d), lambda i: (0, i, 0)),
            pl.BlockSpec((NC, bm, 1), lambda i: (0, i, 0)),
            pl.BlockSpec((d, d), lambda i: (0, 0)),
            pl.BlockSpec((d, d), lambda i: (0, 0)),
        ],
        out_specs=pl.BlockSpec((bm, d), lambda i: (i, 0)),
        out_shape=jax.ShapeDtypeStruct((n_pad, d), jnp.float32),
    )


def kernel(x, weight, edge_index):
    n, d = x.shape
    e = edge_index.shape[1]
    n_pad = -(-n // 2048) * 2048
    src = edge_index[0]
    dst = edge_index[1]

    h0 = jnp.zeros((n_pad, d), jnp.float32).at[:n].set(x)
    sums1, degp = _sc_neighbor_sum(n_pad, d, e, True)(h0, src, dst)
    deg2 = degp.reshape(NC, n_pad, 1)
    tc = _tc_layer(n_pad, d)
    h1 = tc(h0, sums1, deg2, weight[0, :, :d], weight[0, :, d:])
    (sums2,) = _sc_neighbor_sum(n_pad, d, e, False)(h1, src, dst)
    h2 = tc(h1, sums2, deg2, weight[1, :, :d], weight[1, :, d:])
    return h2[:n]


# speedup vs baseline: 4.9663x; 4.9663x over previous
import functools

import jax
import jax.numpy as jnp
from jax import lax
from jax.experimental import pallas as pl
from jax.experimental.pallas import tpu as pltpu
from jax.experimental.pallas import tpu_sc as plsc

NC = 2
NS = 16
LANES = 16
CH = 80


@functools.lru_cache(maxsize=None)
def _sc_neighbor_sum(n_pad: int, din: int, e: int):
    nw = NC * NS
    epw = e // nw
    assert epw * nw == e and epw % CH == 0
    nch = epw // CH
    rows_per_tile = n_pad // NS
    assert rows_per_tile * NS == n_pad
    zrows = 64
    assert rows_per_tile % zrows == 0
    assert din % LANES == 0

    mesh = plsc.VectorSubcoreMesh(
        core_axis_name="c", subcore_axis_name="s",
        num_cores=NC, num_subcores=NS)

    @functools.partial(
        pl.kernel,
        out_type=(jax.ShapeDtypeStruct((NC, n_pad, din), jnp.float32),),
        mesh=mesh,
        compiler_params=pltpu.CompilerParams(use_tc_tiling_on_sc=False),
        scratch_types=[
            pltpu.VMEM((CH,), jnp.int32),
            pltpu.VMEM((CH,), jnp.int32),
            pltpu.VMEM((CH, din), jnp.float32),
            pltpu.VMEM((zrows, din), jnp.float32),
            pltpu.VMEM_SHARED((n_pad, din), jnp.float32),
            pltpu.SemaphoreType.DMA,
        ])
    def sc_kernel(h_hbm, src_hbm, dst_hbm, sum_out,
                  src_v, dst_v, rows_v, zbuf, acc_sh, sem):
        c = lax.axis_index("c")
        s = lax.axis_index("s")
        wid = c * NS + s
        zvec = jnp.zeros((LANES,), jnp.float32)

        def zb_body(i, carry):
            zbuf[i // (din // LANES),
                 pl.ds((i % (din // LANES)) * LANES, LANES)] = zvec
            return carry
        lax.fori_loop(0, zrows * (din // LANES), zb_body, None)

        def za_body(j, carry):
            pltpu.sync_copy(
                zbuf, acc_sh.at[pl.ds(s * rows_per_tile + j * zrows, zrows)])
            return carry
        lax.fori_loop(0, rows_per_tile // zrows, za_body, None)

        plsc.subcore_barrier()

        ebase = wid * epw

        def edge_body(i, carry):
            off = ebase + i * CH
            pltpu.sync_copy(src_hbm.at[pl.ds(off, CH)], src_v)
            pltpu.sync_copy(dst_hbm.at[pl.ds(off, CH)], dst_v)
            pltpu.async_copy(h_hbm.at[src_v], rows_v, sem).wait()
            pltpu.sync_copy(rows_v, acc_sh.at[dst_v], add=True)
            return carry
        lax.fori_loop(0, nch, edge_body, None)

        plsc.subcore_barrier()

        def co_body(j, carry):
            r0 = s * rows_per_tile + j * zrows
            pltpu.sync_copy(acc_sh.at[pl.ds(r0, zrows)], zbuf)
            pltpu.sync_copy(zbuf, sum_out.at[c, pl.ds(r0, zrows)])
            return carry
        lax.fori_loop(0, rows_per_tile // zrows, co_body, None)

    return sc_kernel


@functools.lru_cache(maxsize=None)
def _tc_layer(n_pad: int, d: int, din: int):
    bm = 1024
    assert n_pad % bm == 0

    def body(h_ref, p_ref, deg_ref, ws_ref, wn_ref, o_ref):
        deg = jnp.maximum(deg_ref[0] + deg_ref[1], 1.0)
        p = p_ref[0] + p_ref[1]
        m = p[:, :d] / deg
        h = h_ref[...][:, :d]
        dn = (((1,), (1,)), ((), ()))
        acc = lax.dot_general(h, ws_ref[...], dn,
                              preferred_element_type=jnp.float32,
                              precision=lax.Precision.HIGHEST)
        acc = acc + lax.dot_general(m, wn_ref[...], dn,
                                    preferred_element_type=jnp.float32,
                                    precision=lax.Precision.HIGHEST)
        o_ref[...] = jnp.maximum(acc, 0.0)

    return pl.pallas_call(
        body,
        grid=(n_pad // bm,),
        in_specs=[
            pl.BlockSpec((bm, din), lambda i: (i, 0)),
            pl.BlockSpec((NC, bm, din), lambda i: (0, i, 0)),
            pl.BlockSpec((NC, bm, 1), lambda i: (0, i, 0)),
            pl.BlockSpec((d, d), lambda i: (0, 0)),
            pl.BlockSpec((d, d), lambda i: (0, 0)),
        ],
        out_specs=pl.BlockSpec((bm, d), lambda i: (i, 0)),
        out_shape=jax.ShapeDtypeStruct((n_pad, d), jnp.float32),
    )


def kernel(x, weight, edge_index):
    n, d = x.shape
    e = edge_index.shape[1]
    n_pad = -(-n // 2048) * 2048
    dw = d + LANES
    src = edge_index[0]
    dst = edge_index[1]

    h0w = (jnp.zeros((n_pad, dw), jnp.float32)
           .at[:n, :d].set(x)
           .at[:n, d].set(1.0))
    (sums1,) = _sc_neighbor_sum(n_pad, dw, e)(h0w, src, dst)
    deg2 = sums1[:, :, d:d + 1]
    h1 = _tc_layer(n_pad, d, dw)(h0w, sums1, deg2,
                                 weight[0, :, :d], weight[0, :, d:])
    (sums2,) = _sc_neighbor_sum(n_pad, d, e)(h1, src, dst)
    h2 = _tc_layer(n_pad, d, d)(h1, sums2, deg2,
                                weight[1, :, :d], weight[1, :, d:])
    return h2[:n]


# speedup vs baseline: 7.4426x; 1.4986x over previous
import functools

import jax
import jax.numpy as jnp
from jax import lax
from jax.experimental import pallas as pl
from jax.experimental.pallas import tpu as pltpu
from jax.experimental.pallas import tpu_sc as plsc

NC = 2
NS = 16
LANES = 16
CH = 80


@functools.lru_cache(maxsize=None)
def _sc_neighbor_sum(n_pad: int, din: int, e: int):
    nw = NC * NS
    epw = e // nw
    assert epw * nw == e and epw % CH == 0
    nch = epw // CH
    assert nch % 2 == 1 and nch >= 3
    rows_per_tile = n_pad // NS
    assert rows_per_tile * NS == n_pad
    assert din % LANES == 0

    mesh = plsc.VectorSubcoreMesh(
        core_axis_name="c", subcore_axis_name="s",
        num_cores=NC, num_subcores=NS)

    @functools.partial(
        pl.kernel,
        out_type=(jax.ShapeDtypeStruct((NC, n_pad, din), jnp.float32),),
        mesh=mesh,
        compiler_params=pltpu.CompilerParams(use_tc_tiling_on_sc=False),
        scratch_types=[
            pltpu.VMEM((CH,), jnp.int32),
            pltpu.VMEM((CH,), jnp.int32),
            pltpu.VMEM((CH,), jnp.int32),
            pltpu.VMEM((CH,), jnp.int32),
            pltpu.VMEM((CH, din), jnp.float32),
            pltpu.VMEM((CH, din), jnp.float32),
            pltpu.VMEM_SHARED((n_pad, din), jnp.float32),
            pltpu.SemaphoreType.DMA,
            pltpu.SemaphoreType.DMA,
            pltpu.SemaphoreType.DMA,
        ])
    def sc_kernel(h_hbm, src_hbm, dst_hbm, z_hbm, sum_out,
                  srcA, dstA, srcB, dstB, rowsA, rowsB, acc_sh,
                  semZ, semA, semB):
        c = lax.axis_index("c")
        s = lax.axis_index("s")
        wid = c * NS + s
        ebase = wid * epw
        tile0 = s * rows_per_tile

        zd = pltpu.async_copy(z_hbm, acc_sh.at[pl.ds(tile0, rows_per_tile)],
                              semZ)
        pltpu.sync_copy(src_hbm.at[pl.ds(ebase, CH)], srcA)
        pltpu.sync_copy(dst_hbm.at[pl.ds(ebase, CH)], dstA)
        pltpu.async_copy(h_hbm.at[srcA], rowsA, semA)
        zd.wait()
        plsc.subcore_barrier()

        def body(k, carry):
            i = 2 * k
            offB = ebase + (i + 1) * CH
            pltpu.sync_copy(src_hbm.at[pl.ds(offB, CH)], srcB)
            pltpu.sync_copy(dst_hbm.at[pl.ds(offB, CH)], dstB)
            gB = pltpu.async_copy(h_hbm.at[srcB], rowsB, semB)
            pltpu.make_async_copy(h_hbm.at[srcA], rowsA, semA).wait()
            pltpu.sync_copy(rowsA, acc_sh.at[dstA], add=True)
            offA = ebase + (i + 2) * CH
            pltpu.sync_copy(src_hbm.at[pl.ds(offA, CH)], srcA)
            pltpu.sync_copy(dst_hbm.at[pl.ds(offA, CH)], dstA)
            pltpu.async_copy(h_hbm.at[srcA], rowsA, semA)
            gB.wait()
            pltpu.sync_copy(rowsB, acc_sh.at[dstB], add=True)
            return carry
        lax.fori_loop(0, nch // 2, body, None)

        pltpu.make_async_copy(h_hbm.at[srcA], rowsA, semA).wait()
        pltpu.sync_copy(rowsA, acc_sh.at[dstA], add=True)

        plsc.subcore_barrier()

        pltpu.sync_copy(acc_sh.at[pl.ds(tile0, rows_per_tile)],
                        sum_out.at[c, pl.ds(tile0, rows_per_tile)])

    return sc_kernel


@functools.lru_cache(maxsize=None)
def _tc_layer(n_pad: int, d: int, din: int):
    bm = 1024
    assert n_pad % bm == 0

    def body(h_ref, p_ref, deg_ref, ws_ref, wn_ref, o_ref):
        deg = jnp.maximum(deg_ref[0] + deg_ref[1], 1.0)
        p = p_ref[0] + p_ref[1]
        m = p[:, :d] / deg
        h = h_ref[...][:, :d]
        dn = (((1,), (1,)), ((), ()))
        acc = lax.dot_general(h, ws_ref[...], dn,
                              preferred_element_type=jnp.float32,
                              precision=lax.Precision.HIGHEST)
        acc = acc + lax.dot_general(m, wn_ref[...], dn,
                                    preferred_element_type=jnp.float32,
                                    precision=lax.Precision.HIGHEST)
        o_ref[...] = jnp.maximum(acc, 0.0)

    return pl.pallas_call(
        body,
        grid=(n_pad // bm,),
        in_specs=[
            pl.BlockSpec((bm, din), lambda i: (i, 0)),
            pl.BlockSpec((NC, bm, din), lambda i: (0, i, 0)),
            pl.BlockSpec((NC, bm, 1), lambda i: (0, i, 0)),
            pl.BlockSpec((d, d), lambda i: (0, 0)),
            pl.BlockSpec((d, d), lambda i: (0, 0)),
        ],
        out_specs=pl.BlockSpec((bm, d), lambda i: (i, 0)),
        out_shape=jax.ShapeDtypeStruct((n_pad, d), jnp.float32),
    )


def kernel(x, weight, edge_index):
    n, d = x.shape
    e = edge_index.shape[1]
    n_pad = -(-n // 2048) * 2048
    dw = d + LANES
    src = edge_index[0]
    dst = edge_index[1]

    h0w = (jnp.zeros((n_pad, dw), jnp.float32)
           .at[:n, :d].set(x)
           .at[:n, d].set(1.0))
    zw = jnp.zeros((n_pad // NS, dw), jnp.float32)
    zn = jnp.zeros((n_pad // NS, d), jnp.float32)
    (sums1,) = _sc_neighbor_sum(n_pad, dw, e)(h0w, src, dst, zw)
    deg2 = sums1[:, :, d:d + 1]
    h1 = _tc_layer(n_pad, d, dw)(h0w, sums1, deg2,
                                 weight[0, :, :d], weight[0, :, d:])
    (sums2,) = _sc_neighbor_sum(n_pad, d, e)(h1, src, dst, zn)
    h2 = _tc_layer(n_pad, d, d)(h1, sums2, deg2,
                                weight[1, :, :d], weight[1, :, d:])
    return h2[:n]


# speedup vs baseline: 8.2555x; 1.1092x over previous
import functools

import jax
import jax.numpy as jnp
from jax import lax
from jax.experimental import pallas as pl
from jax.experimental.pallas import tpu as pltpu
from jax.experimental.pallas import tpu_sc as plsc

NC = 2
NS = 16
LANES = 16
CH = 80


@functools.lru_cache(maxsize=None)
def _sc_neighbor_sum(n_pad: int, d: int, e: int, with_deg: bool):
    nw = NC * NS
    epw = e // nw
    assert epw * nw == e and epw % CH == 0
    nch = epw // CH
    assert nch % 2 == 1 and nch >= 3
    rpt = n_pad // NS
    assert rpt * NS == n_pad
    dacc = d + LANES if with_deg else d

    mesh = plsc.VectorSubcoreMesh(
        core_axis_name="c", subcore_axis_name="s",
        num_cores=NC, num_subcores=NS)

    out_type = [jax.ShapeDtypeStruct((NC, n_pad, d), jnp.float32)]
    if with_deg:
        out_type.append(jax.ShapeDtypeStruct((NC, n_pad, LANES), jnp.float32))

    @functools.partial(
        pl.kernel,
        out_type=tuple(out_type),
        mesh=mesh,
        compiler_params=pltpu.CompilerParams(use_tc_tiling_on_sc=False),
        scratch_types=[
            pltpu.VMEM((CH,), jnp.int32),
            pltpu.VMEM((CH,), jnp.int32),
            pltpu.VMEM((CH,), jnp.int32),
            pltpu.VMEM((CH,), jnp.int32),
            pltpu.VMEM((CH, dacc), jnp.float32),
            pltpu.VMEM((CH, dacc), jnp.float32),
            pltpu.VMEM_SHARED((n_pad, dacc), jnp.float32),
            pltpu.SemaphoreType.DMA,
            pltpu.SemaphoreType.DMA,
            pltpu.SemaphoreType.DMA,
        ])
    def sc_kernel(h_hbm, src_hbm, dst_hbm, z_hbm, *rest):
        if with_deg:
            (sum_out, deg_out, srcA, dstA, srcB, dstB, rowsA, rowsB,
             acc_sh, semZ, semA, semB) = rest
        else:
            (sum_out, srcA, dstA, srcB, dstB, rowsA, rowsB,
             acc_sh, semZ, semA, semB) = rest
        c = lax.axis_index("c")
        s = lax.axis_index("s")
        wid = c * NS + s
        ebase = wid * epw
        tile0 = s * rpt

        def gdst(rows):
            return rows

        zd = pltpu.async_copy(z_hbm, acc_sh.at[pl.ds(tile0, rpt)], semZ)
        pltpu.sync_copy(src_hbm.at[pl.ds(ebase, CH)], srcA)
        pltpu.sync_copy(dst_hbm.at[pl.ds(ebase, CH)], dstA)
        pltpu.async_copy(h_hbm.at[srcA], gdst(rowsA), semA)
        zd.wait()
        plsc.subcore_barrier()

        def body(k, carry):
            i = 2 * k
            offB = ebase + (i + 1) * CH
            pltpu.sync_copy(src_hbm.at[pl.ds(offB, CH)], srcB)
            pltpu.sync_copy(dst_hbm.at[pl.ds(offB, CH)], dstB)
            gB = pltpu.async_copy(h_hbm.at[srcB], gdst(rowsB), semB)
            pltpu.make_async_copy(h_hbm.at[srcA], gdst(rowsA), semA).wait()
            pltpu.sync_copy(rowsA, acc_sh.at[dstA], add=True)
            offA = ebase + (i + 2) * CH
            pltpu.sync_copy(src_hbm.at[pl.ds(offA, CH)], srcA)
            pltpu.sync_copy(dst_hbm.at[pl.ds(offA, CH)], dstA)
            pltpu.async_copy(h_hbm.at[srcA], gdst(rowsA), semA)
            gB.wait()
            pltpu.sync_copy(rowsB, acc_sh.at[dstB], add=True)
            return carry
        lax.fori_loop(0, nch // 2, body, None)

        pltpu.make_async_copy(h_hbm.at[srcA], gdst(rowsA), semA).wait()
        pltpu.sync_copy(rowsA, acc_sh.at[dstA], add=True)

        plsc.subcore_barrier()

        if with_deg:
            pltpu.sync_copy(acc_sh.at[pl.ds(tile0, rpt), pl.ds(0, d)],
                            sum_out.at[c, pl.ds(tile0, rpt)])
            pltpu.sync_copy(acc_sh.at[pl.ds(tile0, rpt), pl.ds(d, LANES)],
                            deg_out.at[c, pl.ds(tile0, rpt)])
        else:
            pltpu.sync_copy(acc_sh.at[pl.ds(tile0, rpt)],
                            sum_out.at[c, pl.ds(tile0, rpt)])

    return sc_kernel


@functools.lru_cache(maxsize=None)
def _ones_col(n: int, d: int):
    bm = 1024

    def body(x_ref, o_ref):
        lane = lax.broadcasted_iota(jnp.int32, (bm, LANES), 1)
        tail = jnp.where(lane == 0, 1.0, 0.0).astype(jnp.float32)
        o_ref[...] = jnp.concatenate([x_ref[...], tail], axis=1)

    return pl.pallas_call(
        body,
        grid=(-(-n // bm),),
        in_specs=[pl.BlockSpec((bm, d), lambda i: (i, 0))],
        out_specs=pl.BlockSpec((bm, d + LANES), lambda i: (i, 0)),
        out_shape=jax.ShapeDtypeStruct((n, d + LANES), jnp.float32),
    )


@functools.lru_cache(maxsize=None)
def _tc_layer(n: int, n_pad: int, d: int):
    bm = 1024
    assert n_pad % bm == 0

    def body(h_ref, p_ref, deg_ref, ws_ref, wn_ref, o_ref):
        deg = jnp.maximum(deg_ref[0] + deg_ref[1], 1.0)
        m = (p_ref[0] + p_ref[1]) / deg
        dn = (((1,), (1,)), ((), ()))
        acc = lax.dot_general(h_ref[...], ws_ref[...], dn,
                              preferred_element_type=jnp.float32)
        acc = acc + lax.dot_general(m, wn_ref[...], dn,
                                    preferred_element_type=jnp.float32)
        o_ref[...] = jnp.maximum(acc, 0.0)

    return pl.pallas_call(
        body,
        grid=(n_pad // bm,),
        in_specs=[
            pl.BlockSpec((bm, d), lambda i: (i, 0)),
            pl.BlockSpec((NC, bm, d), lambda i: (0, i, 0)),
            pl.BlockSpec((NC, bm, 1), lambda i: (0, i, 0)),
            pl.BlockSpec((d, d), lambda i: (0, 0)),
            pl.BlockSpec((d, d), lambda i: (0, 0)),
        ],
        out_specs=pl.BlockSpec((bm, d), lambda i: (i, 0)),
        out_shape=jax.ShapeDtypeStruct((n, d), jnp.float32),
    )


def kernel(x, weight, edge_index):
    n, d = x.shape
    e = edge_index.shape[1]
    n_pad = -(-n // 2048) * 2048
    src = edge_index[0]
    dst = edge_index[1]
    rpt = n_pad // NS

    zw = jnp.zeros((rpt, d + LANES), jnp.float32)
    zn = jnp.zeros((rpt, d), jnp.float32)
    h0w = _ones_col(n, d)(x)
    feat1, degp = _sc_neighbor_sum(n_pad, d, e, True)(h0w, src, dst, zw)
    deg2 = degp[:, :, :1]
    tc = _tc_layer(n, n_pad, d)
    h1 = tc(x, feat1, deg2, weight[0, :, :d], weight[0, :, d:])
    (feat2,) = _sc_neighbor_sum(n_pad, d, e, False)(h1, src, dst, zn)
    h2 = tc(h1, feat2, deg2, weight[1, :, :d], weight[1, :, d:])
    return h2
